# baseline (speedup 1.0000x reference)
import jax
import jax.numpy as jnp
from jax.experimental import pallas as pl

_N = 10000
_E = 160000
_R = 90
_CHUNKS = 64


def _seg_softmax(alpha, seg, num_segments):
    amax = jax.ops.segment_max(alpha, seg, num_segments=num_segments)
    amax = jnp.where(jnp.isneginf(amax), 0.0, amax)
    ex = jnp.exp(alpha - amax[seg])
    denom = jax.ops.segment_sum(ex, seg, num_segments=num_segments)
    return ex / (denom[seg] + 1e-16)


def _final_body(h2_ref, wl_ref, bl_ref, o_ref):
    z = h2_ref[...] @ wl_ref[...] + bl_ref[...]
    m = jnp.max(z, axis=-1, keepdims=True)
    ez = jnp.exp(z - m)
    o_ref[...] = z - m - jnp.log(jnp.sum(ez, axis=-1, keepdims=True))


def kernel(x, edge_index, edge_type, edge_attr, W_fc, att1, basis1, q1, k1,
           w_mod1, bias1, weight2, q2, k2, W_edge, e2, W_lin, b_lin):
    src, dst = edge_index[0], edge_index[1]
    E = src.shape[0]
    Ec = E // _CHUNKS
    h = x @ W_fc
    w1 = (att1 @ basis1.reshape(35, -1)).reshape(_R, 16, 40)
    hd = jnp.take(h, dst, axis=0)
    hs = jnp.take(h, src, axis=0)

    def _body1(args):
        et_c, hd_c, hs_c = args
        we_c = jnp.take(w1, et_c, axis=0)
        return (jnp.einsum('ec,ecd->ed', hd_c, we_c),
                jnp.einsum('ec,ecd->ed', hs_c, we_c))

    outi, outj = jax.lax.map(
        _body1, (edge_type.reshape(_CHUNKS, Ec), hd.reshape(_CHUNKS, Ec, 16),
                 hs.reshape(_CHUNKS, Ec, 16)))
    outi = outi.reshape(E, 40)
    outj = outj.reshape(E, 40)
    alpha = (outi @ q1) * (outj @ k1)
    seg = dst * _R + edge_type
    alpha = _seg_softmax(alpha, seg, _N * _R)
    oj = outj.reshape(E, 2, 1, 20)
    hmod = w_mod1 * (oj * jnp.ones((E, 2, 2, 1), dtype=outj.dtype))
    msg = oj * alpha.reshape(E, 2, 2, 1) + hmod
    h = jax.ops.segment_sum(msg, dst, num_segments=_N).reshape(_N, 80) + bias1
    hd2 = jnp.take(h, dst, axis=0).reshape(E, 2, 40)
    hs2 = jnp.take(h, src, axis=0).reshape(E, 2, 40)

    def _body2(args):
        et_c, hd_c, hs_c = args
        w2e_c = jnp.take(weight2, et_c, axis=0)
        return (jnp.einsum('ebc,ebcd->ebd', hd_c, w2e_c),
                jnp.einsum('ebc,ebcd->ebd', hs_c, w2e_c))

    outi, outj = jax.lax.map(
        _body2, (edge_type.reshape(_CHUNKS, Ec),
                 hd2.reshape(_CHUNKS, Ec, 2, 40), hs2.reshape(_CHUNKS, Ec, 2, 40)))
    outi = outi.reshape(E, 50)
    outj = outj.reshape(E, 50)
    alpha_edge = (edge_attr @ W_edge) @ e2
    alpha = jax.nn.leaky_relu(outi @ q2 + outj @ k2 + alpha_edge,
                              negative_slope=0.2)
    alpha = _seg_softmax(alpha, dst, _N)
    msg = alpha.reshape(E, 2, 1) * outj.reshape(E, 2, 25)
    h2 = jax.ops.segment_sum(msg, dst, num_segments=_N).reshape(_N, 50)

    return pl.pallas_call(
        _final_body,
        grid=(10,),
        in_specs=[pl.BlockSpec((1000, 50), lambda i: (i, 0)),
                  pl.BlockSpec((50, 4), lambda i: (0, 0)),
                  pl.BlockSpec((1, 4), lambda i: (0, 0))],
        out_specs=pl.BlockSpec((1000, 4), lambda i: (i, 0)),
        out_shape=jax.ShapeDtypeStruct((_N, 4), jnp.float32),
    )(h2, W_lin, b_lin.reshape(1, 4))


# speedup vs baseline: 1.1228x; 1.1228x over previous
import functools

import jax
import jax.numpy as jnp
from jax.experimental import pallas as pl
from jax.experimental.pallas import tpu as pltpu

_N = 10000
_E = 160000
_R = 90
_B = 256
_NB = _E // _B + _R
_EP = _NB * _B
_NEG = -1e30


def _prep(edge_index, edge_type):
    src, dst = edge_index[0], edge_index[1]
    key = edge_type * _N + dst
    perm = jnp.argsort(key)
    skey = key[perm]
    set_ = edge_type[perm]
    r_ar = jnp.arange(_R, dtype=jnp.int32)
    start_r = jnp.searchsorted(set_, r_ar, side='left').astype(jnp.int32)
    end_r = jnp.searchsorted(set_, r_ar, side='right').astype(jnp.int32)
    cnt_r = end_r - start_r
    nb_r = (cnt_r + _B - 1) // _B
    ends = jnp.cumsum(nb_r)
    j = jnp.arange(_NB, dtype=jnp.int32)
    r_j = jnp.searchsorted(ends, j, side='right').astype(jnp.int32)
    valid_block = r_j < _R
    r_jc = jnp.where(valid_block, r_j, 0)
    lb = j - (ends[r_jc] - nb_r[r_jc])
    blk_valid = jnp.where(valid_block,
                          jnp.clip(cnt_r[r_jc] - lb * _B, 0, _B), 0).astype(jnp.int32)
    blk_first = ((lb == 0) | ~valid_block).astype(jnp.int32)
    blk_relw = r_jc
    blk_relm = jnp.where(valid_block, r_j, _R).astype(jnp.int32)
    p = jnp.arange(_EP, dtype=jnp.int32)
    jj = p // _B
    ss = p % _B
    rr = r_jc[jj]
    i = start_r[rr] + (jj - (ends[rr] - nb_r[rr])) * _B + ss
    valid_slot = ss < blk_valid[jj]
    i_c = jnp.clip(i, 0, _E - 1)
    eid = jnp.where(valid_slot, perm[i_c], 0)
    newseg = jnp.concatenate([jnp.ones((1,), jnp.int32),
                              (skey[1:] != skey[:-1]).astype(jnp.int32)])
    segid_sorted = jnp.cumsum(newseg).astype(jnp.int32) - 1
    seg_slot = jnp.where(valid_slot, segid_sorted[i_c], _EP - 1)
    return dict(eid=eid, src_slot=src[eid], dst_slot=dst[eid],
                rel_slot=rr, seg_slot=seg_slot,
                blk_relw=blk_relw, blk_relm=blk_relm,
                blk_valid=blk_valid, blk_first=blk_first)


def _mm_body(a_ref, b_ref, o_ref):
    o_ref[...] = jnp.dot(a_ref[...], b_ref[...],
                         preferred_element_type=jnp.float32)


def _matmul(a, b, blk_rows):
    m, k = a.shape
    _, n = b.shape
    return pl.pallas_call(
        _mm_body,
        grid=(m // blk_rows,),
        in_specs=[pl.BlockSpec((blk_rows, k), lambda i: (i, 0)),
                  pl.BlockSpec((k, n), lambda i: (0, 0))],
        out_specs=pl.BlockSpec((blk_rows, n), lambda i: (i, 0)),
        out_shape=jax.ShapeDtypeStruct((m, n), jnp.float32),
    )(a, b)


def _wprep_body(att1_ref, b2d_ref, we_ref, e2_ref, w1_ref, we2_ref):
    w1_ref[...] = jnp.dot(att1_ref[...], b2d_ref[...],
                          preferred_element_type=jnp.float32)
    we2_ref[...] = jnp.dot(we_ref[...], e2_ref[...],
                           preferred_element_type=jnp.float32)


def _gmm1_body(relw_ref, relm_ref, valid_ref, first_ref,
               hd_ref, hs_ref, w_ref, q1_ref, k1_ref,
               alpha_ref, oj_ref, mr_ref):
    i = pl.program_id(0)
    v = valid_ref[i]
    w = w_ref[0]
    oi = jnp.dot(hd_ref[...], w, preferred_element_type=jnp.float32)
    oj = jnp.dot(hs_ref[...], w, preferred_element_type=jnp.float32)
    rowmask = jax.lax.broadcasted_iota(jnp.int32, (_B, 1), 0) < v
    oj = jnp.where(rowmask, oj, 0.0)
    aq = jnp.dot(oi, q1_ref[...], preferred_element_type=jnp.float32)
    ak = jnp.dot(oj, k1_ref[...], preferred_element_type=jnp.float32)
    a = jnp.where(rowmask, aq * ak, _NEG)
    for c in range(4):
        alpha_ref[c:c + 1, :] = a[:, c:c + 1].reshape(1, _B)
    oj_ref[0] = oj[:, :20]
    oj_ref[1] = oj[:, 20:]
    mcur = jnp.max(a, axis=0).reshape(1, 4, 1)

    @pl.when(first_ref[i] == 1)
    def _():
        mr_ref[...] = mcur

    @pl.when(first_ref[i] == 0)
    def _():
        mr_ref[...] = jnp.maximum(mr_ref[...], mcur)


def _gmm1(hd1, hs1, w1_3d, q1, k1, blk_relw, blk_relm, blk_valid, blk_first):
    grid_spec = pltpu.PrefetchScalarGridSpec(
        num_scalar_prefetch=4,
        grid=(_NB,),
        in_specs=[
            pl.BlockSpec((_B, 16), lambda i, rw, rm, v, f: (i, 0)),
            pl.BlockSpec((_B, 16), lambda i, rw, rm, v, f: (i, 0)),
            pl.BlockSpec((1, 16, 40), lambda i, rw, rm, v, f: (rw[i], 0, 0)),
            pl.BlockSpec((40, 4), lambda i, rw, rm, v, f: (0, 0)),
            pl.BlockSpec((40, 4), lambda i, rw, rm, v, f: (0, 0)),
        ],
        out_specs=[
            pl.BlockSpec((4, _B), lambda i, rw, rm, v, f: (0, i)),
            pl.BlockSpec((2, _B, 20), lambda i, rw, rm, v, f: (0, i, 0)),
            pl.BlockSpec((1, 4, 1), lambda i, rw, rm, v, f: (rm[i], 0, 0)),
        ],
    )
    return pl.pallas_call(
        _gmm1_body,
        grid_spec=grid_spec,
        out_shape=[jax.ShapeDtypeStruct((4, _EP), jnp.float32),
                   jax.ShapeDtypeStruct((2, _EP, 20), jnp.float32),
                   jax.ShapeDtypeStruct((_R + 1, 4, 1), jnp.float32)],
    )(blk_relw, blk_relm, blk_valid, blk_first, hd1, hs1, w1_3d, q1, k1)


def _gmm2_body(relw_ref, relm_ref, valid_ref, first_ref,
               hd_ref, hs_ref, ea_ref, w2_ref, q2_ref, k2_ref, we2_ref,
               alpha_ref, oj_ref, m2_ref):
    i = pl.program_id(0)
    v = valid_ref[i]
    w0 = w2_ref[0, 0]
    w1 = w2_ref[0, 1]
    oi = jnp.concatenate(
        [jnp.dot(hd_ref[:, :40], w0, preferred_element_type=jnp.float32),
         jnp.dot(hd_ref[:, 40:], w1, preferred_element_type=jnp.float32)],
        axis=1)
    oj = jnp.concatenate(
        [jnp.dot(hs_ref[:, :40], w0, preferred_element_type=jnp.float32),
         jnp.dot(hs_ref[:, 40:], w1, preferred_element_type=jnp.float32)],
        axis=1)
    rowmask = jax.lax.broadcasted_iota(jnp.int32, (_B, 1), 0) < v
    oj = jnp.where(rowmask, oj, 0.0)
    ae = jnp.dot(ea_ref[...], we2_ref[...], preferred_element_type=jnp.float32)
    a = (jnp.dot(oi, q2_ref[...], preferred_element_type=jnp.float32)
         + jnp.dot(oj, k2_ref[...], preferred_element_type=jnp.float32) + ae)
    a = jnp.where(a >= 0.0, a, 0.2 * a)
    a = jnp.where(rowmask, a, _NEG)
    for c in range(2):
        alpha_ref[c:c + 1, :] = a[:, c:c + 1].reshape(1, _B)
    z7 = jnp.zeros((_B, 7), jnp.float32)
    oj_ref[0] = jnp.concatenate([oj[:, :25], z7], axis=1)
    oj_ref[1] = jnp.concatenate([oj[:, 25:], z7], axis=1)
    mcur = jnp.max(a, axis=0).reshape(2, 1)

    @pl.when(i == 0)
    def _():
        m2_ref[...] = mcur

    @pl.when(i != 0)
    def _():
        m2_ref[...] = jnp.maximum(m2_ref[...], mcur)


def _gmm2(hd2, hs2, ea, w2, q2, k2, we2, blk_relw, blk_relm, blk_valid,
          blk_first):
    grid_spec = pltpu.PrefetchScalarGridSpec(
        num_scalar_prefetch=4,
        grid=(_NB,),
        in_specs=[
            pl.BlockSpec((_B, 80), lambda i, rw, rm, v, f: (i, 0)),
            pl.BlockSpec((_B, 80), lambda i, rw, rm, v, f: (i, 0)),
            pl.BlockSpec((_B, 16), lambda i, rw, rm, v, f: (i, 0)),
            pl.BlockSpec((1, 2, 40, 25), lambda i, rw, rm, v, f: (rw[i], 0, 0, 0)),
            pl.BlockSpec((50, 2), lambda i, rw, rm, v, f: (0, 0)),
            pl.BlockSpec((50, 2), lambda i, rw, rm, v, f: (0, 0)),
            pl.BlockSpec((16, 2), lambda i, rw, rm, v, f: (0, 0)),
        ],
        out_specs=[
            pl.BlockSpec((2, _B), lambda i, rw, rm, v, f: (0, i)),
            pl.BlockSpec((2, _B, 32), lambda i, rw, rm, v, f: (0, i, 0)),
            pl.BlockSpec((2, 1), lambda i, rw, rm, v, f: (0, 0)),
        ],
    )
    return pl.pallas_call(
        _gmm2_body,
        grid_spec=grid_spec,
        out_shape=[jax.ShapeDtypeStruct((2, _EP), jnp.float32),
                   jax.ShapeDtypeStruct((2, _EP, 32), jnp.float32),
                   jax.ShapeDtypeStruct((2, 1), jnp.float32)],
    )(blk_relw, blk_relm, blk_valid, blk_first, hd2, hs2, ea, w2, q2, k2, we2)


def _msg1_body(beta_ref, oj_ref, msg_ref):
    b = beta_ref[...]
    z4 = jnp.zeros((beta_ref.shape[0], 4), jnp.float32)
    for c in range(2):
        ojc = oj_ref[c]
        msg_ref[c] = jnp.concatenate(
            [ojc * b[:, 2 * c:2 * c + 1], ojc * b[:, 2 * c + 1:2 * c + 2],
             ojc, z4], axis=1)


def _msg1(beta, oj):
    bs = 1280
    return pl.pallas_call(
        _msg1_body,
        grid=(_EP // bs,),
        in_specs=[pl.BlockSpec((bs, 4), lambda i: (i, 0)),
                  pl.BlockSpec((2, bs, 20), lambda i: (0, i, 0))],
        out_specs=pl.BlockSpec((2, bs, 64), lambda i: (0, i, 0)),
        out_shape=jax.ShapeDtypeStruct((2, _EP, 64), jnp.float32),
    )(beta, oj)


def _msg2_body(beta_ref, oj_ref, msg_ref):
    b = beta_ref[...]
    for c in range(2):
        msg_ref[c] = oj_ref[c] * b[:, c:c + 1]


def _msg2(beta2, oj2):
    bs = 1280
    return pl.pallas_call(
        _msg2_body,
        grid=(_EP // bs,),
        in_specs=[pl.BlockSpec((bs, 2), lambda i: (i, 0)),
                  pl.BlockSpec((2, bs, 32), lambda i: (0, i, 0))],
        out_specs=pl.BlockSpec((2, bs, 32), lambda i: (0, i, 0)),
        out_shape=jax.ShapeDtypeStruct((2, _EP, 32), jnp.float32),
    )(beta2, oj2)


def _h1_body(acc_ref, wmod_ref, bias_ref, h1_ref):
    wm = wmod_ref[...]
    parts = []
    for hd in range(2):
        t = acc_ref[hd]
        mod = t[:, 40:60] * wm
        parts.append(t[:, :40] + jnp.concatenate([mod, mod], axis=1))
    h1_ref[...] = jnp.concatenate(parts, axis=1) + bias_ref[...]


def _h1_assemble(acc, w_mod1, bias1):
    bn = 1000
    return pl.pallas_call(
        _h1_body,
        grid=(_N // bn,),
        in_specs=[pl.BlockSpec((2, bn, 64), lambda i: (0, i, 0)),
                  pl.BlockSpec((1, 20), lambda i: (0, 0)),
                  pl.BlockSpec((1, 80), lambda i: (0, 0))],
        out_specs=pl.BlockSpec((bn, 80), lambda i: (i, 0)),
        out_shape=jax.ShapeDtypeStruct((_N, 80), jnp.float32),
    )(acc, w_mod1.reshape(1, 20), bias1.reshape(1, 80))


def _final_body(acc_ref, wl_ref, bl_ref, o_ref):
    h2 = jnp.concatenate([acc_ref[0][:, :25], acc_ref[1][:, :25]], axis=1)
    z = jnp.dot(h2, wl_ref[...], preferred_element_type=jnp.float32) + bl_ref[...]
    m = jnp.max(z, axis=-1, keepdims=True)
    ez = jnp.exp(z - m)
    o_ref[...] = z - m - jnp.log(jnp.sum(ez, axis=-1, keepdims=True))


def _final(acc2, W_lin, b_lin):
    bn = 1000
    return pl.pallas_call(
        _final_body,
        grid=(_N // bn,),
        in_specs=[pl.BlockSpec((2, bn, 32), lambda i: (0, i, 0)),
                  pl.BlockSpec((50, 4), lambda i: (0, 0)),
                  pl.BlockSpec((1, 4), lambda i: (0, 0))],
        out_specs=pl.BlockSpec((bn, 4), lambda i: (i, 0)),
        out_shape=jax.ShapeDtypeStruct((_N, 4), jnp.float32),
    )(acc2, W_lin, b_lin.reshape(1, 4))


def kernel(x, edge_index, edge_type, edge_attr, W_fc, att1, basis1, q1, k1,
           w_mod1, bias1, weight2, q2, k2, W_edge, e2, W_lin, b_lin):
    pr = _prep(edge_index, edge_type)
    src_slot, dst_slot = pr['src_slot'], pr['dst_slot']
    rel_slot, seg_slot, eid = pr['rel_slot'], pr['seg_slot'], pr['eid']

    w1_flat, we2 = pl.pallas_call(
        _wprep_body,
        in_specs=[pl.BlockSpec((_R, 35), lambda: (0, 0)),
                  pl.BlockSpec((35, 640), lambda: (0, 0)),
                  pl.BlockSpec((16, 50), lambda: (0, 0)),
                  pl.BlockSpec((50, 2), lambda: (0, 0))],
        out_specs=[pl.BlockSpec((_R, 640), lambda: (0, 0)),
                   pl.BlockSpec((16, 2), lambda: (0, 0))],
        out_shape=[jax.ShapeDtypeStruct((_R, 640), jnp.float32),
                   jax.ShapeDtypeStruct((16, 2), jnp.float32)],
    )(att1, basis1.reshape(35, 640), W_edge, e2)
    w1_3d = w1_flat.reshape(_R, 16, 40)
    h = _matmul(x, W_fc, 1000)

    hd1 = jnp.take(h, dst_slot, axis=0)
    hs1 = jnp.take(h, src_slot, axis=0)
    ea = jnp.take(edge_attr, eid, axis=0)
    alpha, oj, mr3 = _gmm1(hd1, hs1, w1_3d, q1, k1, pr['blk_relw'],
                           pr['blk_relm'], pr['blk_valid'], pr['blk_first'])
    mr = mr3.reshape(_R + 1, 4).T
    ex = jnp.exp(alpha - mr[:, rel_slot])
    denom = jax.ops.segment_sum(ex.T, seg_slot, num_segments=_EP)
    beta = (ex.T / (denom[seg_slot] + 1e-16))
    msg = _msg1(beta, oj)
    acc = jax.vmap(lambda m: jax.ops.segment_sum(m, dst_slot, num_segments=_N))(msg)
    h1 = _h1_assemble(acc, w_mod1, bias1)

    hd2 = jnp.take(h1, dst_slot, axis=0)
    hs2 = jnp.take(h1, src_slot, axis=0)
    alpha2, oj2, m2 = _gmm2(hd2, hs2, ea, weight2, q2, k2, we2,
                            pr['blk_relw'], pr['blk_relm'], pr['blk_valid'],
                            pr['blk_first'])
    ex2 = jnp.exp(alpha2 - m2)
    denom2 = jax.ops.segment_sum(ex2.T, dst_slot, num_segments=_N)
    beta2 = (ex2.T / (denom2[dst_slot] + 1e-16))
    msg2 = _msg2(beta2, oj2)
    acc2 = jax.vmap(lambda m: jax.ops.segment_sum(m, dst_slot, num_segments=_N))(msg2)
    return _final(acc2, W_lin, b_lin)


# speedup vs baseline: 1.1244x; 1.0014x over previous
import functools

import jax
import jax.numpy as jnp
from jax import lax
from jax.experimental import pallas as pl
from jax.experimental.pallas import tpu as pltpu
from jax.experimental.pallas import tpu_sc as plsc

_N = 10000
_E = 160000
_R = 90
_B = 256
_NB = _E // _B + _R
_EP = _NB * _B
_NEG = -1e30


def _prep(edge_index, edge_type):
    src, dst = edge_index[0], edge_index[1]
    key = edge_type * _N + dst
    perm = jnp.argsort(key)
    skey = key[perm]
    set_ = edge_type[perm]
    r_ar = jnp.arange(_R, dtype=jnp.int32)
    start_r = jnp.searchsorted(set_, r_ar, side='left').astype(jnp.int32)
    end_r = jnp.searchsorted(set_, r_ar, side='right').astype(jnp.int32)
    cnt_r = end_r - start_r
    nb_r = (cnt_r + _B - 1) // _B
    ends = jnp.cumsum(nb_r)
    j = jnp.arange(_NB, dtype=jnp.int32)
    r_j = jnp.searchsorted(ends, j, side='right').astype(jnp.int32)
    valid_block = r_j < _R
    r_jc = jnp.where(valid_block, r_j, 0)
    lb = j - (ends[r_jc] - nb_r[r_jc])
    blk_valid = jnp.where(valid_block,
                          jnp.clip(cnt_r[r_jc] - lb * _B, 0, _B), 0).astype(jnp.int32)
    blk_first = ((lb == 0) | ~valid_block).astype(jnp.int32)
    blk_relw = r_jc
    blk_relm = jnp.where(valid_block, r_j, _R).astype(jnp.int32)
    p = jnp.arange(_EP, dtype=jnp.int32)
    jj = p // _B
    ss = p % _B
    rr = r_jc[jj]
    i = start_r[rr] + (jj - (ends[rr] - nb_r[rr])) * _B + ss
    valid_slot = ss < blk_valid[jj]
    i_c = jnp.clip(i, 0, _E - 1)
    eid = jnp.where(valid_slot, perm[i_c], 0)
    newseg = jnp.concatenate([jnp.ones((1,), jnp.int32),
                              (skey[1:] != skey[:-1]).astype(jnp.int32)])
    segid_sorted = jnp.cumsum(newseg).astype(jnp.int32) - 1
    seg_slot = jnp.where(valid_slot, segid_sorted[i_c], _EP - 1)
    return dict(eid=eid, src_slot=src[eid], dst_slot=dst[eid],
                rel_slot=rr, seg_slot=seg_slot,
                blk_relw=blk_relw, blk_relm=blk_relm,
                blk_valid=blk_valid, blk_first=blk_first)


def _mm_body(a_ref, b_ref, o_ref):
    o_ref[...] = jnp.dot(a_ref[...], b_ref[...],
                         preferred_element_type=jnp.float32)


def _matmul(a, b, blk_rows):
    m, k = a.shape
    _, n = b.shape
    return pl.pallas_call(
        _mm_body,
        grid=(m // blk_rows,),
        in_specs=[pl.BlockSpec((blk_rows, k), lambda i: (i, 0)),
                  pl.BlockSpec((k, n), lambda i: (0, 0))],
        out_specs=pl.BlockSpec((blk_rows, n), lambda i: (i, 0)),
        out_shape=jax.ShapeDtypeStruct((m, n), jnp.float32),
    )(a, b)


def _wprep_body(att1_ref, b2d_ref, we_ref, e2_ref, w1_ref, we2_ref):
    w1_ref[...] = jnp.dot(att1_ref[...], b2d_ref[...],
                          preferred_element_type=jnp.float32)
    we2_ref[...] = jnp.dot(we_ref[...], e2_ref[...],
                           preferred_element_type=jnp.float32)


def _gmm1_body(relw_ref, relm_ref, valid_ref, first_ref,
               hd_ref, hs_ref, w_ref, q1_ref, k1_ref,
               alpha_ref, oj_ref, mr_ref):
    i = pl.program_id(0)
    v = valid_ref[i]
    w = w_ref[0]
    oi = jnp.dot(hd_ref[...], w, preferred_element_type=jnp.float32)
    oj = jnp.dot(hs_ref[...], w, preferred_element_type=jnp.float32)
    rowmask = jax.lax.broadcasted_iota(jnp.int32, (_B, 1), 0) < v
    oj = jnp.where(rowmask, oj, 0.0)
    aq = jnp.dot(oi, q1_ref[...], preferred_element_type=jnp.float32)
    ak = jnp.dot(oj, k1_ref[...], preferred_element_type=jnp.float32)
    a = jnp.where(rowmask, aq * ak, _NEG)
    alpha_ref[...] = a
    oj_ref[0] = oj[:, :20]
    oj_ref[1] = oj[:, 20:]
    mcur = jnp.max(a, axis=0).reshape(1, 1, 4)

    @pl.when(first_ref[i] == 1)
    def _():
        mr_ref[...] = mcur

    @pl.when(first_ref[i] == 0)
    def _():
        mr_ref[...] = jnp.maximum(mr_ref[...], mcur)


def _gmm1(hd1, hs1, w1_3d, q1, k1, blk_relw, blk_relm, blk_valid, blk_first):
    grid_spec = pltpu.PrefetchScalarGridSpec(
        num_scalar_prefetch=4,
        grid=(_NB,),
        in_specs=[
            pl.BlockSpec((_B, 16), lambda i, rw, rm, v, f: (i, 0)),
            pl.BlockSpec((_B, 16), lambda i, rw, rm, v, f: (i, 0)),
            pl.BlockSpec((1, 16, 40), lambda i, rw, rm, v, f: (rw[i], 0, 0)),
            pl.BlockSpec((40, 4), lambda i, rw, rm, v, f: (0, 0)),
            pl.BlockSpec((40, 4), lambda i, rw, rm, v, f: (0, 0)),
        ],
        out_specs=[
            pl.BlockSpec((_B, 4), lambda i, rw, rm, v, f: (i, 0)),
            pl.BlockSpec((2, _B, 20), lambda i, rw, rm, v, f: (0, i, 0)),
            pl.BlockSpec((1, 1, 4), lambda i, rw, rm, v, f: (rm[i], 0, 0)),
        ],
    )
    return pl.pallas_call(
        _gmm1_body,
        grid_spec=grid_spec,
        out_shape=[jax.ShapeDtypeStruct((_EP, 4), jnp.float32),
                   jax.ShapeDtypeStruct((2, _EP, 20), jnp.float32),
                   jax.ShapeDtypeStruct((_R + 1, 1, 4), jnp.float32)],
    )(blk_relw, blk_relm, blk_valid, blk_first, hd1, hs1, w1_3d, q1, k1)


def _gmm2_body(relw_ref, relm_ref, valid_ref, first_ref,
               hd_ref, hs_ref, ea_ref, w2_ref, q2_ref, k2_ref, we2_ref,
               alpha_ref, oj_ref, m2_ref):
    i = pl.program_id(0)
    v = valid_ref[i]
    w0 = w2_ref[0, 0]
    w1 = w2_ref[0, 1]
    oi = jnp.concatenate(
        [jnp.dot(hd_ref[:, :40], w0, preferred_element_type=jnp.float32),
         jnp.dot(hd_ref[:, 40:], w1, preferred_element_type=jnp.float32)],
        axis=1)
    oj = jnp.concatenate(
        [jnp.dot(hs_ref[:, :40], w0, preferred_element_type=jnp.float32),
         jnp.dot(hs_ref[:, 40:], w1, preferred_element_type=jnp.float32)],
        axis=1)
    rowmask = jax.lax.broadcasted_iota(jnp.int32, (_B, 1), 0) < v
    oj = jnp.where(rowmask, oj, 0.0)
    ae = jnp.dot(ea_ref[...], we2_ref[...], preferred_element_type=jnp.float32)
    a = (jnp.dot(oi, q2_ref[...], preferred_element_type=jnp.float32)
         + jnp.dot(oj, k2_ref[...], preferred_element_type=jnp.float32) + ae)
    a = jnp.where(a >= 0.0, a, 0.2 * a)
    a = jnp.where(rowmask, a, _NEG)
    alpha_ref[...] = a
    z7 = jnp.zeros((_B, 7), jnp.float32)
    oj_ref[0] = jnp.concatenate([oj[:, :25], z7], axis=1)
    oj_ref[1] = jnp.concatenate([oj[:, 25:], z7], axis=1)
    mcur = jnp.max(a, axis=0).reshape(2, 1) + jnp.zeros((2, 16), jnp.float32)

    @pl.when(i == 0)
    def _():
        m2_ref[...] = mcur

    @pl.when(i != 0)
    def _():
        m2_ref[...] = jnp.maximum(m2_ref[...], mcur)


def _gmm2(hd2, hs2, ea, w2, q2, k2, we2, blk_relw, blk_relm, blk_valid,
          blk_first):
    grid_spec = pltpu.PrefetchScalarGridSpec(
        num_scalar_prefetch=4,
        grid=(_NB,),
        in_specs=[
            pl.BlockSpec((_B, 80), lambda i, rw, rm, v, f: (i, 0)),
            pl.BlockSpec((_B, 80), lambda i, rw, rm, v, f: (i, 0)),
            pl.BlockSpec((_B, 16), lambda i, rw, rm, v, f: (i, 0)),
            pl.BlockSpec((1, 2, 40, 25), lambda i, rw, rm, v, f: (rw[i], 0, 0, 0)),
            pl.BlockSpec((50, 2), lambda i, rw, rm, v, f: (0, 0)),
            pl.BlockSpec((50, 2), lambda i, rw, rm, v, f: (0, 0)),
            pl.BlockSpec((16, 2), lambda i, rw, rm, v, f: (0, 0)),
        ],
        out_specs=[
            pl.BlockSpec((_B, 2), lambda i, rw, rm, v, f: (i, 0)),
            pl.BlockSpec((2, _B, 32), lambda i, rw, rm, v, f: (0, i, 0)),
            pl.BlockSpec((2, 16), lambda i, rw, rm, v, f: (0, 0)),
        ],
    )
    return pl.pallas_call(
        _gmm2_body,
        grid_spec=grid_spec,
        out_shape=[jax.ShapeDtypeStruct((_EP, 2), jnp.float32),
                   jax.ShapeDtypeStruct((2, _EP, 32), jnp.float32),
                   jax.ShapeDtypeStruct((2, 16), jnp.float32)],
    )(blk_relw, blk_relm, blk_valid, blk_first, hd2, hs2, ea, w2, q2, k2, we2)


def _msg1_body(ex_ref, den_ref, oj_ref, msg_ref):
    b = ex_ref[...] / (den_ref[...] + 1e-16)
    z4 = jnp.zeros((ex_ref.shape[0], 4), jnp.float32)
    for c in range(2):
        ojc = oj_ref[c]
        msg_ref[c] = jnp.concatenate(
            [ojc * b[:, 2 * c:2 * c + 1], ojc * b[:, 2 * c + 1:2 * c + 2],
             ojc, z4], axis=1)


def _msg1(ex, den, oj):
    bs = 1280
    return pl.pallas_call(
        _msg1_body,
        grid=(_EP // bs,),
        in_specs=[pl.BlockSpec((bs, 4), lambda i: (i, 0)),
                  pl.BlockSpec((bs, 4), lambda i: (i, 0)),
                  pl.BlockSpec((2, bs, 20), lambda i: (0, i, 0))],
        out_specs=pl.BlockSpec((2, bs, 64), lambda i: (0, i, 0)),
        out_shape=jax.ShapeDtypeStruct((2, _EP, 64), jnp.float32),
    )(ex, den, oj)


def _msg2_body(ex_ref, den_ref, oj_ref, msg_ref):
    b = ex_ref[...] / (den_ref[...] + 1e-16)
    for c in range(2):
        msg_ref[c] = oj_ref[c] * b[:, c:c + 1]


def _msg2(ex2, den2, oj2):
    bs = 1280
    return pl.pallas_call(
        _msg2_body,
        grid=(_EP // bs,),
        in_specs=[pl.BlockSpec((bs, 2), lambda i: (i, 0)),
                  pl.BlockSpec((bs, 2), lambda i: (i, 0)),
                  pl.BlockSpec((2, bs, 32), lambda i: (0, i, 0))],
        out_specs=pl.BlockSpec((2, bs, 32), lambda i: (0, i, 0)),
        out_shape=jax.ShapeDtypeStruct((2, _EP, 32), jnp.float32),
    )(ex2, den2, oj2)


def _mesh():
    return plsc.VectorSubcoreMesh(core_axis_name="c", subcore_axis_name="s")
_GC = _EP // 32
_TCH = _EP // 16
_NS = _N // 16


def _sc_gather2(tab, idxa, idxb):
    d = tab.shape[1]

    def body(tab_h, ia_h, ib_h, oa_h, ob_h, idxv, rbuf, sem):
        cc = lax.axis_index("c")
        ss = lax.axis_index("s")
        wid = ss * 2 + cc
        base = wid * _GC
        for src_h, out_h in ((ia_h, oa_h), (ib_h, ob_h)):
            pltpu.sync_copy(src_h.at[wid], idxv)

            def outer(o, _):
                for q in range(5):
                    pltpu.async_copy(tab_h.at[idxv.at[o * 5 + q]],
                                     rbuf.at[pl.ds(q * 104, 104), :],
                                     sem).wait()
                pltpu.sync_copy(rbuf, out_h.at[pl.ds(base + o * 520, 520), :])
                return 0

            lax.fori_loop(0, 11, outer, 0)

    f = pl.kernel(
        body,
        out_type=[jax.ShapeDtypeStruct((_EP, d), jnp.float32),
                  jax.ShapeDtypeStruct((_EP, d), jnp.float32)],
        mesh=_mesh(),
        compiler_params=pltpu.CompilerParams(use_tc_tiling_on_sc=False),
        scratch_types=[pltpu.VMEM((55, 104), jnp.int32),
                       pltpu.VMEM((520, d), jnp.float32),
                       pltpu.SemaphoreType.DMA],
    )
    return f(tab, idxa.reshape(32, 55, 104), idxb.reshape(32, 55, 104))


def _sc_gather1(tab, idx):
    d = tab.shape[1]

    def body(tab_h, ia_h, oa_h, idxv, rbuf, sem):
        cc = lax.axis_index("c")
        ss = lax.axis_index("s")
        wid = ss * 2 + cc
        base = wid * _GC
        pltpu.sync_copy(ia_h.at[wid], idxv)

        def outer(o, _):
            for q in range(5):
                pltpu.async_copy(tab_h.at[idxv.at[o * 5 + q]],
                                 rbuf.at[pl.ds(q * 104, 104), :], sem).wait()
            pltpu.sync_copy(rbuf, oa_h.at[pl.ds(base + o * 520, 520), :])
            return 0

        lax.fori_loop(0, 11, outer, 0)

    f = pl.kernel(
        body,
        out_type=jax.ShapeDtypeStruct((_EP, d), jnp.float32),
        mesh=_mesh(),
        compiler_params=pltpu.CompilerParams(use_tc_tiling_on_sc=False),
        scratch_types=[pltpu.VMEM((55, 104), jnp.int32),
                       pltpu.VMEM((520, d), jnp.float32),
                       pltpu.SemaphoreType.DMA],
    )
    return f(tab, idx.reshape(32, 55, 104))


def _sm1(alpha, mrp, rel_slot, seg_slot):
    def body(al_h, mrp_h, rel_h, seg_h, exs_h, den_h,
             abuf, rbuf, sbuf, ebuf, mrv, zbuf, dent, sem):
        cc = lax.axis_index("c")
        ss = lax.axis_index("s")
        base = ss * _TCH
        iot = lax.iota(jnp.int32, 16)
        zero16 = jnp.zeros((16,), jnp.float32)
        for w in range(5):
            for k in range(2):
                plsc.store_scatter(zbuf, [w * 16 + iot, iot * 0 + k], zero16)

        def _z(q, _):
            pltpu.sync_copy(zbuf, dent.at[pl.ds(base + q * 80, 80), :])
            return 0

        lax.fori_loop(0, 143, _z, 0)
        pltpu.sync_copy(al_h.at[pl.ds(base, _TCH), :], abuf)
        pltpu.sync_copy(rel_h.at[ss], rbuf)
        pltpu.sync_copy(seg_h.at[ss], sbuf)
        pltpu.sync_copy(mrp_h, mrv)
        plsc.subcore_barrier()

        def _p1(p, _):
            def _w(w, __):
                lrow = w * 16 + iot
                grow = p * 80 + lrow
                relv = plsc.load_gather(rbuf, [iot * 0 + p, lrow])
                for k in range(2):
                    comp = iot * 0 + (2 * cc + k)
                    mval = plsc.load_gather(mrv, [comp, relv])
                    av = plsc.load_gather(abuf, [grow, comp])
                    exv = jnp.exp(av - mval)
                    plsc.store_scatter(ebuf, [grow, iot * 0 + k], exv)
                return 0

            lax.fori_loop(0, 5, _w, 0)
            pltpu.sync_copy(ebuf.at[pl.ds(p * 80, 80), :],
                            dent.at[sbuf.at[p]], add=True)
            return 0

        lax.fori_loop(0, 143, _p1, 0)
        pltpu.sync_copy(ebuf, exs_h.at[pl.ds(base, _TCH), pl.ds(2 * cc, 2)])
        plsc.subcore_barrier()

        def _p2(p, _):
            pltpu.async_copy(dent.at[sbuf.at[p]],
                             ebuf.at[pl.ds(p * 80, 80), :], sem).wait()
            return 0

        lax.fori_loop(0, 143, _p2, 0)
        pltpu.sync_copy(ebuf, den_h.at[pl.ds(base, _TCH), pl.ds(2 * cc, 2)])

    f = pl.kernel(
        body,
        out_type=[jax.ShapeDtypeStruct((_EP, 4), jnp.float32),
                  jax.ShapeDtypeStruct((_EP, 4), jnp.float32)],
        mesh=_mesh(),
        compiler_params=pltpu.CompilerParams(use_tc_tiling_on_sc=False),
        scratch_types=[pltpu.VMEM((_TCH, 4), jnp.float32),
                       pltpu.VMEM((143, 80), jnp.int32),
                       pltpu.VMEM((143, 80), jnp.int32),
                       pltpu.VMEM((_TCH, 2), jnp.float32),
                       pltpu.VMEM((4, 96), jnp.float32),
                       pltpu.VMEM((80, 2), jnp.float32),
                       pltpu.VMEM_SHARED((_EP, 2), jnp.float32),
                       pltpu.SemaphoreType.DMA],
    )
    return f(alpha, mrp, rel_slot.reshape(16, 143, 80),
             seg_slot.reshape(16, 143, 80))


def _sm2(alpha2, m2f, dst_slot):
    def body(al_h, m2_h, dst_h, exs_h, den_h,
             abuf, dbuf, ebuf, mb, zbuf, dent, sem):
        cc = lax.axis_index("c")
        ss = lax.axis_index("s")
        base = ss * _TCH
        iot = lax.iota(jnp.int32, 16)
        zero16 = jnp.zeros((16,), jnp.float32)
        for w in range(5):
            for k in range(2):
                plsc.store_scatter(zbuf, [w * 16 + iot, iot * 0 + k], zero16)

        def _z(q, _):
            pltpu.sync_copy(zbuf.at[pl.ds(0, 25), :],
                            dent.at[pl.ds(ss * _NS + q * 25, 25), :])
            return 0

        lax.fori_loop(0, 25, _z, 0)
        pltpu.sync_copy(al_h.at[pl.ds(base, _TCH), :], abuf)
        pltpu.sync_copy(dst_h.at[ss], dbuf)
        pltpu.sync_copy(m2_h.at[pl.ds(cc * 16, 16)], mb)
        plsc.subcore_barrier()
        mval = mb[...]

        def _p1(p, _):
            def _w(w, __):
                lrow = w * 16 + iot
                grow = p * 80 + lrow
                av = plsc.load_gather(abuf, [grow, iot * 0 + cc])
                exv = jnp.exp(av - mval)
                for k in range(2):
                    plsc.store_scatter(ebuf, [grow, iot * 0 + k], exv)
                return 0

            lax.fori_loop(0, 5, _w, 0)
            pltpu.sync_copy(ebuf.at[pl.ds(p * 80, 80), :],
                            dent.at[dbuf.at[p]], add=True)
            return 0

        lax.fori_loop(0, 143, _p1, 0)
        pltpu.sync_copy(ebuf.at[:, 0:1],
                        exs_h.at[pl.ds(base, _TCH), pl.ds(cc, 1)])
        plsc.subcore_barrier()

        def _p2(p, _):
            pltpu.async_copy(dent.at[dbuf.at[p]],
                             ebuf.at[pl.ds(p * 80, 80), :], sem).wait()
            return 0

        lax.fori_loop(0, 143, _p2, 0)
        pltpu.sync_copy(ebuf.at[:, 0:1],
                        den_h.at[pl.ds(base, _TCH), pl.ds(cc, 1)])

    f = pl.kernel(
        body,
        out_type=[jax.ShapeDtypeStruct((_EP, 2), jnp.float32),
                  jax.ShapeDtypeStruct((_EP, 2), jnp.float32)],
        mesh=_mesh(),
        compiler_params=pltpu.CompilerParams(use_tc_tiling_on_sc=False),
        scratch_types=[pltpu.VMEM((_TCH, 2), jnp.float32),
                       pltpu.VMEM((143, 80), jnp.int32),
                       pltpu.VMEM((_TCH, 2), jnp.float32),
                       pltpu.VMEM((16,), jnp.float32),
                       pltpu.VMEM((80, 2), jnp.float32),
                       pltpu.VMEM_SHARED((_N, 2), jnp.float32),
                       pltpu.SemaphoreType.DMA],
    )
    return f(alpha2, m2f, dst_slot.reshape(16, 143, 80))


def _sc_scatter(msg, dst_slot):
    d = msg.shape[2]
    zr = 125 * d // 16

    def body(msg_h, idx_h, acc_h, idxv, mbuf, zbuf, acct, sem):
        cc = lax.axis_index("c")
        ss = lax.axis_index("s")
        base = ss * _TCH
        iot = lax.iota(jnp.int32, 16)
        zero16 = jnp.zeros((16,), jnp.float32)
        for w in range(zr):
            fl = w * 16 + iot
            plsc.store_scatter(zbuf, [fl // d, fl % d], zero16)

        def _z(q, _):
            pltpu.sync_copy(zbuf, acct.at[pl.ds(ss * _NS + q * 125, 125), :])
            return 0

        lax.fori_loop(0, 5, _z, 0)
        pltpu.sync_copy(idx_h.at[ss], idxv)
        plsc.subcore_barrier()

        def _p(p, _):
            pltpu.sync_copy(msg_h.at[cc, pl.ds(base + p * 104, 104), :], mbuf)
            pltpu.sync_copy(mbuf, acct.at[idxv.at[p]], add=True)
            return 0

        lax.fori_loop(0, 110, _p, 0)
        plsc.subcore_barrier()

        def _d(q, _):
            pltpu.sync_copy(acct.at[pl.ds(ss * _NS + q * 125, 125), :], zbuf)
            pltpu.sync_copy(zbuf, acc_h.at[cc, pl.ds(ss * _NS + q * 125, 125), :])
            return 0

        lax.fori_loop(0, 5, _d, 0)

    f = pl.kernel(
        body,
        out_type=jax.ShapeDtypeStruct((2, _N, d), jnp.float32),
        mesh=_mesh(),
        compiler_params=pltpu.CompilerParams(use_tc_tiling_on_sc=False),
        scratch_types=[pltpu.VMEM((110, 104), jnp.int32),
                       pltpu.VMEM((104, d), jnp.float32),
                       pltpu.VMEM((125, d), jnp.float32),
                       pltpu.VMEM_SHARED((_N, d), jnp.float32),
                       pltpu.SemaphoreType.DMA],
    )
    return f(msg, dst_slot.reshape(16, 110, 104))


def _h1_body(acc_ref, wmod_ref, bias_ref, h1_ref):
    wm = wmod_ref[...]
    parts = []
    for hd in range(2):
        t = acc_ref[hd]
        mod = t[:, 40:60] * wm
        parts.append(t[:, :40] + jnp.concatenate([mod, mod], axis=1))
    h1_ref[...] = jnp.concatenate(parts, axis=1) + bias_ref[...]


def _h1_assemble(acc, w_mod1, bias1):
    bn = 1000
    return pl.pallas_call(
        _h1_body,
        grid=(_N // bn,),
        in_specs=[pl.BlockSpec((2, bn, 64), lambda i: (0, i, 0)),
                  pl.BlockSpec((1, 20), lambda i: (0, 0)),
                  pl.BlockSpec((1, 80), lambda i: (0, 0))],
        out_specs=pl.BlockSpec((bn, 80), lambda i: (i, 0)),
        out_shape=jax.ShapeDtypeStruct((_N, 80), jnp.float32),
    )(acc, w_mod1.reshape(1, 20), bias1.reshape(1, 80))


def _final_body(acc_ref, wl_ref, bl_ref, o_ref):
    h2 = jnp.concatenate([acc_ref[0][:, :25], acc_ref[1][:, :25]], axis=1)
    z = jnp.dot(h2, wl_ref[...], preferred_element_type=jnp.float32) + bl_ref[...]
    m = jnp.max(z, axis=-1, keepdims=True)
    ez = jnp.exp(z - m)
    o_ref[...] = z - m - jnp.log(jnp.sum(ez, axis=-1, keepdims=True))


def _final(acc2, W_lin, b_lin):
    bn = 1000
    return pl.pallas_call(
        _final_body,
        grid=(_N // bn,),
        in_specs=[pl.BlockSpec((2, bn, 32), lambda i: (0, i, 0)),
                  pl.BlockSpec((50, 4), lambda i: (0, 0)),
                  pl.BlockSpec((1, 4), lambda i: (0, 0))],
        out_specs=pl.BlockSpec((bn, 4), lambda i: (i, 0)),
        out_shape=jax.ShapeDtypeStruct((_N, 4), jnp.float32),
    )(acc2, W_lin, b_lin.reshape(1, 4))


def kernel(x, edge_index, edge_type, edge_attr, W_fc, att1, basis1, q1, k1,
           w_mod1, bias1, weight2, q2, k2, W_edge, e2, W_lin, b_lin):
    pr = _prep(edge_index, edge_type)
    src_slot, dst_slot = pr['src_slot'], pr['dst_slot']
    rel_slot, seg_slot, eid = pr['rel_slot'], pr['seg_slot'], pr['eid']

    w1_flat, we2 = pl.pallas_call(
        _wprep_body,
        in_specs=[pl.BlockSpec((_R, 35), lambda: (0, 0)),
                  pl.BlockSpec((35, 640), lambda: (0, 0)),
                  pl.BlockSpec((16, 50), lambda: (0, 0)),
                  pl.BlockSpec((50, 2), lambda: (0, 0))],
        out_specs=[pl.BlockSpec((_R, 640), lambda: (0, 0)),
                   pl.BlockSpec((16, 2), lambda: (0, 0))],
        out_shape=[jax.ShapeDtypeStruct((_R, 640), jnp.float32),
                   jax.ShapeDtypeStruct((16, 2), jnp.float32)],
    )(att1, basis1.reshape(35, 640), W_edge, e2)
    w1_3d = w1_flat.reshape(_R, 16, 40)
    h = _matmul(x, W_fc, 1000)

    hd1 = jnp.take(h, dst_slot, axis=0)
    hs1 = jnp.take(h, src_slot, axis=0)
    ea = jnp.take(edge_attr, eid, axis=0)
    alpha, oj, mr3 = _gmm1(hd1, hs1, w1_3d, q1, k1, pr['blk_relw'],
                           pr['blk_relm'], pr['blk_valid'], pr['blk_first'])
    mr = mr3.reshape(_R + 1, 4)
    exs = jnp.exp(alpha - mr[rel_slot])
    denom = jax.ops.segment_sum(exs, seg_slot, num_segments=_EP)
    dens = denom[seg_slot]
    msg = _msg1(exs, dens, oj)
    acc = jax.vmap(lambda m: jax.ops.segment_sum(m, dst_slot, num_segments=_N))(msg)
    h1 = _h1_assemble(acc, w_mod1, bias1)

    hd2 = jnp.take(h1, dst_slot, axis=0)
    hs2 = jnp.take(h1, src_slot, axis=0)
    alpha2, oj2, m2 = _gmm2(hd2, hs2, ea, weight2, q2, k2, we2,
                            pr['blk_relw'], pr['blk_relm'], pr['blk_valid'],
                            pr['blk_first'])
    ex2 = jnp.exp(alpha2 - m2[:, 0][None, :])
    denom2 = jax.ops.segment_sum(ex2, dst_slot, num_segments=_N)
    den2 = denom2[dst_slot]
    msg2 = _msg2(ex2, den2, oj2)
    acc2 = jax.vmap(lambda m: jax.ops.segment_sum(m, dst_slot, num_segments=_N))(msg2)
    return _final(acc2, W_lin, b_lin)


# speedup vs baseline: 1.1363x; 1.0106x over previous
import functools

import jax
import jax.numpy as jnp
from jax import lax
from jax.experimental import pallas as pl
from jax.experimental.pallas import tpu as pltpu
from jax.experimental.pallas import tpu_sc as plsc

_N = 10000
_E = 160000
_R = 90
_B = 256
_NB = _E // _B + _R
_EP = _NB * _B
_NEG = -1e30


def _prep(edge_index, edge_type):
    src, dst = edge_index[0], edge_index[1]
    key = edge_type * _N + dst
    eidx = jnp.arange(_E, dtype=jnp.int32)
    skey, s_src, s_dst, s_eid = lax.sort((key, src, dst, eidx), num_keys=1)
    set_ = skey // _N
    r_ar = jnp.arange(_R, dtype=jnp.int32)
    start_r = jnp.searchsorted(set_, r_ar, side='left').astype(jnp.int32)
    end_r = jnp.searchsorted(set_, r_ar, side='right').astype(jnp.int32)
    cnt_r = end_r - start_r
    nb_r = (cnt_r + _B - 1) // _B
    ends = jnp.cumsum(nb_r).astype(jnp.int32)
    j = jnp.arange(_NB, dtype=jnp.int32)
    r_j = jnp.searchsorted(ends, j, side='right').astype(jnp.int32)
    valid_block = r_j < _R
    r_jc = jnp.where(valid_block, r_j, 0)
    lb = j - (ends[r_jc] - nb_r[r_jc])
    blk_valid = jnp.where(valid_block,
                          jnp.clip(cnt_r[r_jc] - lb * _B, 0, _B), 0).astype(jnp.int32)
    blk_first = ((lb == 0) | ~valid_block).astype(jnp.int32)
    blk_relw = r_jc
    blk_relm = jnp.where(valid_block, r_j, _R).astype(jnp.int32)
    first_same = jnp.searchsorted(set_, set_, side='left').astype(jnp.int32)
    onehot = (set_[:, None] == r_ar[None, :]).astype(jnp.float32)
    padstart_r = ((ends - nb_r) * _B).astype(jnp.float32)
    padstart_e = jnp.dot(onehot, padstart_r,
                         precision=lax.Precision.HIGHEST).astype(jnp.int32)
    pos = eidx - first_same + padstart_e
    newseg = jnp.concatenate([jnp.ones((1,), jnp.int32),
                              (skey[1:] != skey[:-1]).astype(jnp.int32)])
    segid_sorted = jnp.cumsum(newseg).astype(jnp.int32) - 1
    src0 = src[0]
    dst0 = dst[0]
    eid = jnp.zeros((_EP,), jnp.int32).at[pos].set(s_eid)
    src_slot = jnp.full((_EP,), src0, jnp.int32).at[pos].set(s_src)
    dst_slot = jnp.full((_EP,), dst0, jnp.int32).at[pos].set(s_dst)
    seg_slot = jnp.full((_EP,), _EP - 1, jnp.int32).at[pos].set(segid_sorted)
    rel_slot = jnp.repeat(r_jc, _B, total_repeat_length=_EP)
    return dict(eid=eid, src_slot=src_slot, dst_slot=dst_slot,
                rel_slot=rel_slot, seg_slot=seg_slot,
                blk_relw=blk_relw, blk_relm=blk_relm,
                blk_valid=blk_valid, blk_first=blk_first)


def _mm_body(a_ref, b_ref, o_ref):
    o_ref[...] = jnp.dot(a_ref[...], b_ref[...],
                         preferred_element_type=jnp.float32)


def _matmul(a, b, blk_rows):
    m, k = a.shape
    _, n = b.shape
    return pl.pallas_call(
        _mm_body,
        grid=(m // blk_rows,),
        in_specs=[pl.BlockSpec((blk_rows, k), lambda i: (i, 0)),
                  pl.BlockSpec((k, n), lambda i: (0, 0))],
        out_specs=pl.BlockSpec((blk_rows, n), lambda i: (i, 0)),
        out_shape=jax.ShapeDtypeStruct((m, n), jnp.float32),
    )(a, b)


def _wprep_body(att1_ref, b2d_ref, we_ref, e2_ref, w1_ref, we2_ref):
    w1_ref[...] = jnp.dot(att1_ref[...], b2d_ref[...],
                          preferred_element_type=jnp.float32)
    we2_ref[...] = jnp.dot(we_ref[...], e2_ref[...],
                           preferred_element_type=jnp.float32)


def _gmm1_body(relw_ref, relm_ref, valid_ref, first_ref,
               hd_ref, hs_ref, w_ref, q1_ref, k1_ref,
               alpha_ref, oj_ref, mr_ref):
    i = pl.program_id(0)
    v = valid_ref[i]
    w = w_ref[0]
    oi = jnp.dot(hd_ref[...], w, preferred_element_type=jnp.float32)
    oj = jnp.dot(hs_ref[...], w, preferred_element_type=jnp.float32)
    rowmask = jax.lax.broadcasted_iota(jnp.int32, (_B, 1), 0) < v
    oj = jnp.where(rowmask, oj, 0.0)
    aq = jnp.dot(oi, q1_ref[...], preferred_element_type=jnp.float32)
    ak = jnp.dot(oj, k1_ref[...], preferred_element_type=jnp.float32)
    a = jnp.where(rowmask, aq * ak, _NEG)
    alpha_ref[...] = a
    oj_ref[0] = oj[:, :20]
    oj_ref[1] = oj[:, 20:]
    mcur = jnp.max(a, axis=0).reshape(1, 1, 4)

    @pl.when(first_ref[i] == 1)
    def _():
        mr_ref[...] = mcur

    @pl.when(first_ref[i] == 0)
    def _():
        mr_ref[...] = jnp.maximum(mr_ref[...], mcur)


def _gmm1(hd1, hs1, w1_3d, q1, k1, blk_relw, blk_relm, blk_valid, blk_first):
    grid_spec = pltpu.PrefetchScalarGridSpec(
        num_scalar_prefetch=4,
        grid=(_NB,),
        in_specs=[
            pl.BlockSpec((_B, 16), lambda i, rw, rm, v, f: (i, 0)),
            pl.BlockSpec((_B, 16), lambda i, rw, rm, v, f: (i, 0)),
            pl.BlockSpec((1, 16, 40), lambda i, rw, rm, v, f: (rw[i], 0, 0)),
            pl.BlockSpec((40, 4), lambda i, rw, rm, v, f: (0, 0)),
            pl.BlockSpec((40, 4), lambda i, rw, rm, v, f: (0, 0)),
        ],
        out_specs=[
            pl.BlockSpec((_B, 4), lambda i, rw, rm, v, f: (i, 0)),
            pl.BlockSpec((2, _B, 20), lambda i, rw, rm, v, f: (0, i, 0)),
            pl.BlockSpec((1, 1, 4), lambda i, rw, rm, v, f: (rm[i], 0, 0)),
        ],
    )
    return pl.pallas_call(
        _gmm1_body,
        grid_spec=grid_spec,
        out_shape=[jax.ShapeDtypeStruct((_EP, 4), jnp.float32),
                   jax.ShapeDtypeStruct((2, _EP, 20), jnp.float32),
                   jax.ShapeDtypeStruct((_R + 1, 1, 4), jnp.float32)],
    )(blk_relw, blk_relm, blk_valid, blk_first, hd1, hs1, w1_3d, q1, k1)


def _gmm2_body(relw_ref, relm_ref, valid_ref, first_ref,
               hd_ref, hs_ref, ea_ref, w2_ref, q2_ref, k2_ref, we2_ref,
               alpha_ref, oj_ref, m2_ref):
    i = pl.program_id(0)
    v = valid_ref[i]
    w0 = w2_ref[0, 0]
    w1 = w2_ref[0, 1]
    oi = jnp.concatenate(
        [jnp.dot(hd_ref[:, :40], w0, preferred_element_type=jnp.float32),
         jnp.dot(hd_ref[:, 40:], w1, preferred_element_type=jnp.float32)],
        axis=1)
    oj = jnp.concatenate(
        [jnp.dot(hs_ref[:, :40], w0, preferred_element_type=jnp.float32),
         jnp.dot(hs_ref[:, 40:], w1, preferred_element_type=jnp.float32)],
        axis=1)
    rowmask = jax.lax.broadcasted_iota(jnp.int32, (_B, 1), 0) < v
    oj = jnp.where(rowmask, oj, 0.0)
    ae = jnp.dot(ea_ref[...], we2_ref[...], preferred_element_type=jnp.float32)
    a = (jnp.dot(oi, q2_ref[...], preferred_element_type=jnp.float32)
         + jnp.dot(oj, k2_ref[...], preferred_element_type=jnp.float32) + ae)
    a = jnp.where(a >= 0.0, a, 0.2 * a)
    a = jnp.where(rowmask, a, _NEG)
    alpha_ref[...] = a
    z7 = jnp.zeros((_B, 7), jnp.float32)
    oj_ref[0] = jnp.concatenate([oj[:, :25], z7], axis=1)
    oj_ref[1] = jnp.concatenate([oj[:, 25:], z7], axis=1)
    mcur = jnp.max(a, axis=0).reshape(2, 1) + jnp.zeros((2, 16), jnp.float32)

    @pl.when(i == 0)
    def _():
        m2_ref[...] = mcur

    @pl.when(i != 0)
    def _():
        m2_ref[...] = jnp.maximum(m2_ref[...], mcur)


def _gmm2(hd2, hs2, ea, w2, q2, k2, we2, blk_relw, blk_relm, blk_valid,
          blk_first):
    grid_spec = pltpu.PrefetchScalarGridSpec(
        num_scalar_prefetch=4,
        grid=(_NB,),
        in_specs=[
            pl.BlockSpec((_B, 80), lambda i, rw, rm, v, f: (i, 0)),
            pl.BlockSpec((_B, 80), lambda i, rw, rm, v, f: (i, 0)),
            pl.BlockSpec((_B, 16), lambda i, rw, rm, v, f: (i, 0)),
            pl.BlockSpec((1, 2, 40, 25), lambda i, rw, rm, v, f: (rw[i], 0, 0, 0)),
            pl.BlockSpec((50, 2), lambda i, rw, rm, v, f: (0, 0)),
            pl.BlockSpec((50, 2), lambda i, rw, rm, v, f: (0, 0)),
            pl.BlockSpec((16, 2), lambda i, rw, rm, v, f: (0, 0)),
        ],
        out_specs=[
            pl.BlockSpec((_B, 2), lambda i, rw, rm, v, f: (i, 0)),
            pl.BlockSpec((2, _B, 32), lambda i, rw, rm, v, f: (0, i, 0)),
            pl.BlockSpec((2, 16), lambda i, rw, rm, v, f: (0, 0)),
        ],
    )
    return pl.pallas_call(
        _gmm2_body,
        grid_spec=grid_spec,
        out_shape=[jax.ShapeDtypeStruct((_EP, 2), jnp.float32),
                   jax.ShapeDtypeStruct((2, _EP, 32), jnp.float32),
                   jax.ShapeDtypeStruct((2, 16), jnp.float32)],
    )(blk_relw, blk_relm, blk_valid, blk_first, hd2, hs2, ea, w2, q2, k2, we2)


def _msg1_body(ex_ref, den_ref, oj_ref, msg_ref):
    b = ex_ref[...] / (den_ref[...] + 1e-16)
    z4 = jnp.zeros((ex_ref.shape[0], 4), jnp.float32)
    for c in range(2):
        ojc = oj_ref[c]
        msg_ref[c] = jnp.concatenate(
            [ojc * b[:, 2 * c:2 * c + 1], ojc * b[:, 2 * c + 1:2 * c + 2],
             ojc, z4], axis=1)


def _msg1(ex, den, oj):
    bs = 1280
    return pl.pallas_call(
        _msg1_body,
        grid=(_EP // bs,),
        in_specs=[pl.BlockSpec((bs, 4), lambda i: (i, 0)),
                  pl.BlockSpec((bs, 4), lambda i: (i, 0)),
                  pl.BlockSpec((2, bs, 20), lambda i: (0, i, 0))],
        out_specs=pl.BlockSpec((2, bs, 64), lambda i: (0, i, 0)),
        out_shape=jax.ShapeDtypeStruct((2, _EP, 64), jnp.float32),
    )(ex, den, oj)


def _msg2_body(ex_ref, den_ref, oj_ref, msg_ref):
    b = ex_ref[...] / (den_ref[...] + 1e-16)
    for c in range(2):
        msg_ref[c] = oj_ref[c] * b[:, c:c + 1]


def _msg2(ex2, den2, oj2):
    bs = 1280
    return pl.pallas_call(
        _msg2_body,
        grid=(_EP // bs,),
        in_specs=[pl.BlockSpec((bs, 2), lambda i: (i, 0)),
                  pl.BlockSpec((bs, 2), lambda i: (i, 0)),
                  pl.BlockSpec((2, bs, 32), lambda i: (0, i, 0))],
        out_specs=pl.BlockSpec((2, bs, 32), lambda i: (0, i, 0)),
        out_shape=jax.ShapeDtypeStruct((2, _EP, 32), jnp.float32),
    )(ex2, den2, oj2)


def _mesh():
    return plsc.VectorSubcoreMesh(core_axis_name="c", subcore_axis_name="s")
_GC = _EP // 32
_TCH = _EP // 16
_NS = _N // 16


def _sc_gather2(tab, idxa, idxb):
    d = tab.shape[1]

    def body(tab_h, ia_h, ib_h, oa_h, ob_h, idxv, rbuf, sem):
        cc = lax.axis_index("c")
        ss = lax.axis_index("s")
        wid = ss * 2 + cc
        base = wid * _GC
        for src_h, out_h in ((ia_h, oa_h), (ib_h, ob_h)):
            pltpu.sync_copy(src_h.at[wid], idxv)

            def outer(o, _):
                for q in range(5):
                    pltpu.async_copy(tab_h.at[idxv.at[o * 5 + q]],
                                     rbuf.at[pl.ds(q * 104, 104), :],
                                     sem).wait()
                pltpu.sync_copy(rbuf, out_h.at[pl.ds(base + o * 520, 520), :])
                return 0

            lax.fori_loop(0, 11, outer, 0)

    f = pl.kernel(
        body,
        out_type=[jax.ShapeDtypeStruct((_EP, d), jnp.float32),
                  jax.ShapeDtypeStruct((_EP, d), jnp.float32)],
        mesh=_mesh(),
        compiler_params=pltpu.CompilerParams(use_tc_tiling_on_sc=False),
        scratch_types=[pltpu.VMEM((55, 104), jnp.int32),
                       pltpu.VMEM((520, d), jnp.float32),
                       pltpu.SemaphoreType.DMA],
    )
    return f(tab, idxa.reshape(32, 55, 104), idxb.reshape(32, 55, 104))


def _sc_gather1(tab, idx):
    d = tab.shape[1]

    def body(tab_h, ia_h, oa_h, idxv, rbuf, sem):
        cc = lax.axis_index("c")
        ss = lax.axis_index("s")
        wid = ss * 2 + cc
        base = wid * _GC
        pltpu.sync_copy(ia_h.at[wid], idxv)

        def outer(o, _):
            for q in range(5):
                pltpu.async_copy(tab_h.at[idxv.at[o * 5 + q]],
                                 rbuf.at[pl.ds(q * 104, 104), :], sem).wait()
            pltpu.sync_copy(rbuf, oa_h.at[pl.ds(base + o * 520, 520), :])
            return 0

        lax.fori_loop(0, 11, outer, 0)

    f = pl.kernel(
        body,
        out_type=jax.ShapeDtypeStruct((_EP, d), jnp.float32),
        mesh=_mesh(),
        compiler_params=pltpu.CompilerParams(use_tc_tiling_on_sc=False),
        scratch_types=[pltpu.VMEM((55, 104), jnp.int32),
                       pltpu.VMEM((520, d), jnp.float32),
                       pltpu.SemaphoreType.DMA],
    )
    return f(tab, idx.reshape(32, 55, 104))


def _sm1(alpha, mrp, rel_slot, seg_slot):
    def body(al_h, mrp_h, rel_h, seg_h, exs_h, den_h,
             abuf, rbuf, sbuf, ebuf, mrv, zbuf, dent, sem):
        cc = lax.axis_index("c")
        ss = lax.axis_index("s")
        base = ss * _TCH
        iot = lax.iota(jnp.int32, 16)
        zero16 = jnp.zeros((16,), jnp.float32)
        for w in range(5):
            for k in range(2):
                plsc.store_scatter(zbuf, [w * 16 + iot, iot * 0 + k], zero16)

        def _z(q, _):
            pltpu.sync_copy(zbuf, dent.at[pl.ds(base + q * 80, 80), :])
            return 0

        lax.fori_loop(0, 143, _z, 0)
        pltpu.sync_copy(al_h.at[pl.ds(base, _TCH), :], abuf)
        pltpu.sync_copy(rel_h.at[ss], rbuf)
        pltpu.sync_copy(seg_h.at[ss], sbuf)
        pltpu.sync_copy(mrp_h, mrv)
        plsc.subcore_barrier()

        def _p1(p, _):
            def _w(w, __):
                lrow = w * 16 + iot
                grow = p * 80 + lrow
                relv = plsc.load_gather(rbuf, [iot * 0 + p, lrow])
                for k in range(2):
                    comp = iot * 0 + (2 * cc + k)
                    mval = plsc.load_gather(mrv, [comp, relv])
                    av = plsc.load_gather(abuf, [grow, comp])
                    exv = jnp.exp(av - mval)
                    plsc.store_scatter(ebuf, [grow, iot * 0 + k], exv)
                return 0

            lax.fori_loop(0, 5, _w, 0)
            pltpu.sync_copy(ebuf.at[pl.ds(p * 80, 80), :],
                            dent.at[sbuf.at[p]], add=True)
            return 0

        lax.fori_loop(0, 143, _p1, 0)
        pltpu.sync_copy(ebuf, exs_h.at[pl.ds(base, _TCH), pl.ds(2 * cc, 2)])
        plsc.subcore_barrier()

        def _p2(p, _):
            pltpu.async_copy(dent.at[sbuf.at[p]],
                             ebuf.at[pl.ds(p * 80, 80), :], sem).wait()
            return 0

        lax.fori_loop(0, 143, _p2, 0)
        pltpu.sync_copy(ebuf, den_h.at[pl.ds(base, _TCH), pl.ds(2 * cc, 2)])

    f = pl.kernel(
        body,
        out_type=[jax.ShapeDtypeStruct((_EP, 4), jnp.float32),
                  jax.ShapeDtypeStruct((_EP, 4), jnp.float32)],
        mesh=_mesh(),
        compiler_params=pltpu.CompilerParams(use_tc_tiling_on_sc=False),
        scratch_types=[pltpu.VMEM((_TCH, 4), jnp.float32),
                       pltpu.VMEM((143, 80), jnp.int32),
                       pltpu.VMEM((143, 80), jnp.int32),
                       pltpu.VMEM((_TCH, 2), jnp.float32),
                       pltpu.VMEM((4, 96), jnp.float32),
                       pltpu.VMEM((80, 2), jnp.float32),
                       pltpu.VMEM_SHARED((_EP, 2), jnp.float32),
                       pltpu.SemaphoreType.DMA],
    )
    return f(alpha, mrp, rel_slot.reshape(16, 143, 80),
             seg_slot.reshape(16, 143, 80))


def _sm2(alpha2, m2f, dst_slot):
    def body(al_h, m2_h, dst_h, exs_h, den_h,
             abuf, dbuf, ebuf, mb, zbuf, dent, sem):
        cc = lax.axis_index("c")
        ss = lax.axis_index("s")
        base = ss * _TCH
        iot = lax.iota(jnp.int32, 16)
        zero16 = jnp.zeros((16,), jnp.float32)
        for w in range(5):
            for k in range(2):
                plsc.store_scatter(zbuf, [w * 16 + iot, iot * 0 + k], zero16)

        def _z(q, _):
            pltpu.sync_copy(zbuf.at[pl.ds(0, 25), :],
                            dent.at[pl.ds(ss * _NS + q * 25, 25), :])
            return 0

        lax.fori_loop(0, 25, _z, 0)
        pltpu.sync_copy(al_h.at[pl.ds(base, _TCH), :], abuf)
        pltpu.sync_copy(dst_h.at[ss], dbuf)
        pltpu.sync_copy(m2_h.at[pl.ds(cc * 16, 16)], mb)
        plsc.subcore_barrier()
        mval = mb[...]

        def _p1(p, _):
            def _w(w, __):
                lrow = w * 16 + iot
                grow = p * 80 + lrow
                av = plsc.load_gather(abuf, [grow, iot * 0 + cc])
                exv = jnp.exp(av - mval)
                for k in range(2):
                    plsc.store_scatter(ebuf, [grow, iot * 0 + k], exv)
                return 0

            lax.fori_loop(0, 5, _w, 0)
            pltpu.sync_copy(ebuf.at[pl.ds(p * 80, 80), :],
                            dent.at[dbuf.at[p]], add=True)
            return 0

        lax.fori_loop(0, 143, _p1, 0)
        pltpu.sync_copy(ebuf.at[:, 0:1],
                        exs_h.at[pl.ds(base, _TCH), pl.ds(cc, 1)])
        plsc.subcore_barrier()

        def _p2(p, _):
            pltpu.async_copy(dent.at[dbuf.at[p]],
                             ebuf.at[pl.ds(p * 80, 80), :], sem).wait()
            return 0

        lax.fori_loop(0, 143, _p2, 0)
        pltpu.sync_copy(ebuf.at[:, 0:1],
                        den_h.at[pl.ds(base, _TCH), pl.ds(cc, 1)])

    f = pl.kernel(
        body,
        out_type=[jax.ShapeDtypeStruct((_EP, 2), jnp.float32),
                  jax.ShapeDtypeStruct((_EP, 2), jnp.float32)],
        mesh=_mesh(),
        compiler_params=pltpu.CompilerParams(use_tc_tiling_on_sc=False),
        scratch_types=[pltpu.VMEM((_TCH, 2), jnp.float32),
                       pltpu.VMEM((143, 80), jnp.int32),
                       pltpu.VMEM((_TCH, 2), jnp.float32),
                       pltpu.VMEM((16,), jnp.float32),
                       pltpu.VMEM((80, 2), jnp.float32),
                       pltpu.VMEM_SHARED((_N, 2), jnp.float32),
                       pltpu.SemaphoreType.DMA],
    )
    return f(alpha2, m2f, dst_slot.reshape(16, 143, 80))


def _sc_scatter(msg, dst_slot):
    d = msg.shape[2]
    zr = 125 * d // 16

    def body(msg_h, idx_h, acc_h, idxv, mbuf, zbuf, acct, sem):
        cc = lax.axis_index("c")
        ss = lax.axis_index("s")
        base = ss * _TCH
        iot = lax.iota(jnp.int32, 16)
        zero16 = jnp.zeros((16,), jnp.float32)
        for w in range(zr):
            fl = w * 16 + iot
            plsc.store_scatter(zbuf, [fl // d, fl % d], zero16)

        def _z(q, _):
            pltpu.sync_copy(zbuf, acct.at[pl.ds(ss * _NS + q * 125, 125), :])
            return 0

        lax.fori_loop(0, 5, _z, 0)
        pltpu.sync_copy(idx_h.at[ss], idxv)
        plsc.subcore_barrier()

        def _p(p, _):
            pltpu.sync_copy(msg_h.at[cc, pl.ds(base + p * 104, 104), :], mbuf)
            pltpu.sync_copy(mbuf, acct.at[idxv.at[p]], add=True)
            return 0

        lax.fori_loop(0, 110, _p, 0)
        plsc.subcore_barrier()

        def _d(q, _):
            pltpu.sync_copy(acct.at[pl.ds(ss * _NS + q * 125, 125), :], zbuf)
            pltpu.sync_copy(zbuf, acc_h.at[cc, pl.ds(ss * _NS + q * 125, 125), :])
            return 0

        lax.fori_loop(0, 5, _d, 0)

    f = pl.kernel(
        body,
        out_type=jax.ShapeDtypeStruct((2, _N, d), jnp.float32),
        mesh=_mesh(),
        compiler_params=pltpu.CompilerParams(use_tc_tiling_on_sc=False),
        scratch_types=[pltpu.VMEM((110, 104), jnp.int32),
                       pltpu.VMEM((104, d), jnp.float32),
                       pltpu.VMEM((125, d), jnp.float32),
                       pltpu.VMEM_SHARED((_N, d), jnp.float32),
                       pltpu.SemaphoreType.DMA],
    )
    return f(msg, dst_slot.reshape(16, 110, 104))


def _h1_body(acc_ref, wmod_ref, bias_ref, h1_ref):
    wm = wmod_ref[...]
    parts = []
    for hd in range(2):
        t = acc_ref[hd]
        mod = t[:, 40:60] * wm
        parts.append(t[:, :40] + jnp.concatenate([mod, mod], axis=1))
    h1_ref[...] = jnp.concatenate(parts, axis=1) + bias_ref[...]


def _h1_assemble(acc, w_mod1, bias1):
    bn = 1000
    return pl.pallas_call(
        _h1_body,
        grid=(_N // bn,),
        in_specs=[pl.BlockSpec((2, bn, 64), lambda i: (0, i, 0)),
                  pl.BlockSpec((1, 20), lambda i: (0, 0)),
                  pl.BlockSpec((1, 80), lambda i: (0, 0))],
        out_specs=pl.BlockSpec((bn, 80), lambda i: (i, 0)),
        out_shape=jax.ShapeDtypeStruct((_N, 80), jnp.float32),
    )(acc, w_mod1.reshape(1, 20), bias1.reshape(1, 80))


def _final_body(acc_ref, wl_ref, bl_ref, o_ref):
    h2 = jnp.concatenate([acc_ref[0][:, :25], acc_ref[1][:, :25]], axis=1)
    z = jnp.dot(h2, wl_ref[...], preferred_element_type=jnp.float32) + bl_ref[...]
    m = jnp.max(z, axis=-1, keepdims=True)
    ez = jnp.exp(z - m)
    o_ref[...] = z - m - jnp.log(jnp.sum(ez, axis=-1, keepdims=True))


def _final(acc2, W_lin, b_lin):
    bn = 1000
    return pl.pallas_call(
        _final_body,
        grid=(_N // bn,),
        in_specs=[pl.BlockSpec((2, bn, 32), lambda i: (0, i, 0)),
                  pl.BlockSpec((50, 4), lambda i: (0, 0)),
                  pl.BlockSpec((1, 4), lambda i: (0, 0))],
        out_specs=pl.BlockSpec((bn, 4), lambda i: (i, 0)),
        out_shape=jax.ShapeDtypeStruct((_N, 4), jnp.float32),
    )(acc2, W_lin, b_lin.reshape(1, 4))


def kernel(x, edge_index, edge_type, edge_attr, W_fc, att1, basis1, q1, k1,
           w_mod1, bias1, weight2, q2, k2, W_edge, e2, W_lin, b_lin):
    pr = _prep(edge_index, edge_type)
    src_slot, dst_slot = pr['src_slot'], pr['dst_slot']
    rel_slot, seg_slot, eid = pr['rel_slot'], pr['seg_slot'], pr['eid']

    w1_flat, we2 = pl.pallas_call(
        _wprep_body,
        in_specs=[pl.BlockSpec((_R, 35), lambda: (0, 0)),
                  pl.BlockSpec((35, 640), lambda: (0, 0)),
                  pl.BlockSpec((16, 50), lambda: (0, 0)),
                  pl.BlockSpec((50, 2), lambda: (0, 0))],
        out_specs=[pl.BlockSpec((_R, 640), lambda: (0, 0)),
                   pl.BlockSpec((16, 2), lambda: (0, 0))],
        out_shape=[jax.ShapeDtypeStruct((_R, 640), jnp.float32),
                   jax.ShapeDtypeStruct((16, 2), jnp.float32)],
    )(att1, basis1.reshape(35, 640), W_edge, e2)
    w1_3d = w1_flat.reshape(_R, 16, 40)
    h = _matmul(x, W_fc, 1000)

    hd1 = jnp.take(h, dst_slot, axis=0)
    hs1 = jnp.take(h, src_slot, axis=0)
    ea = jnp.take(edge_attr, eid, axis=0)
    alpha, oj, mr3 = _gmm1(hd1, hs1, w1_3d, q1, k1, pr['blk_relw'],
                           pr['blk_relm'], pr['blk_valid'], pr['blk_first'])
    mr = mr3.reshape(_R + 1, 4)
    exs = jnp.exp(alpha - mr[rel_slot])
    denom = jax.ops.segment_sum(exs, seg_slot, num_segments=_EP)
    dens = denom[seg_slot]
    msg = _msg1(exs, dens, oj)
    acc = jax.vmap(lambda m: jax.ops.segment_sum(m, dst_slot, num_segments=_N))(msg)
    h1 = _h1_assemble(acc, w_mod1, bias1)

    hd2 = jnp.take(h1, dst_slot, axis=0)
    hs2 = jnp.take(h1, src_slot, axis=0)
    alpha2, oj2, m2 = _gmm2(hd2, hs2, ea, weight2, q2, k2, we2,
                            pr['blk_relw'], pr['blk_relm'], pr['blk_valid'],
                            pr['blk_first'])
    ex2 = jnp.exp(alpha2 - m2[:, 0][None, :])
    denom2 = jax.ops.segment_sum(ex2, dst_slot, num_segments=_N)
    den2 = denom2[dst_slot]
    msg2 = _msg2(ex2, den2, oj2)
    acc2 = jax.vmap(lambda m: jax.ops.segment_sum(m, dst_slot, num_segments=_N))(msg2)
    return _final(acc2, W_lin, b_lin)
